# baseline (speedup 1.0000x reference)
import functools

import jax
import jax.numpy as jnp
from jax import lax
from jax.experimental import pallas as pl
from jax.experimental.pallas import tpu as pltpu
from jax.experimental.pallas import tpu_sc as plsc

_B = 128
_V = 100000
_C = 10000
_K = _V // _C

_NW = 32
_VPW = _V // _NW
_RC = 125
_NCH = _VPW // _RC
_CW = _RC * _B

_GUMBEL_T = jax.random.gumbel(jax.random.key(42), (_B, _V), dtype=jnp.float32).T


def _tc_body(y_ref, mask_ref, gp_ref, m_sc, s_sc):
    p = pl.program_id(0)
    k = pl.program_id(1)
    neg_inf = jnp.float32(-jnp.inf)

    @pl.when((p == 0) & (k == 0))
    def _init():
        m_sc[...] = jnp.full((1, _B), neg_inf, jnp.float32)
        s_sc[...] = jnp.zeros((1, _B), jnp.float32)

    @pl.when(p == 0)
    def _pass_maxsum():
        y = y_ref[...]
        cmax = jnp.max(y, axis=0, keepdims=True)
        m_new = jnp.maximum(m_sc[...], cmax)
        s_sc[...] = (s_sc[...] * jnp.exp(m_sc[...] - m_new)
                     + jnp.sum(jnp.exp(y - m_new), axis=0, keepdims=True))
        m_sc[...] = m_new

    @pl.when(p == 1)
    def _pass_emit():
        gp_ref[...] = (jnp.exp(y_ref[...] - m_sc[...])
                       * (1.0 / s_sc[...]) * mask_ref[...])


def _tc_goldprobs(y_T, mask_T):
    chunk = pl.BlockSpec((_C, _B), lambda p, k: (k, 0))
    chunk_p1 = pl.BlockSpec((_C, _B), lambda p, k: (p * k, 0))
    return pl.pallas_call(
        _tc_body,
        grid=(2, _K),
        in_specs=[chunk, chunk_p1],
        out_specs=chunk_p1,
        out_shape=jax.ShapeDtypeStruct((_V, _B), jnp.float32),
        scratch_shapes=[
            pltpu.VMEM((1, _B), jnp.float32),
            pltpu.VMEM((1, _B), jnp.float32),
        ],
    )(y_T, mask_T)


def _sc_sample_body(y_hbm, m_hbm, g_hbm, s_out, i_out,
                    yb0, yb1, mb0, mb1, gb0, gb1, sbuf, ibuf,
                    sem0, sem1):
    wid = lax.axis_index("c") * 16 + lax.axis_index("s")
    base = wid * _VPW
    ybufs, mbufs, gbufs = (yb0, yb1), (mb0, mb1), (gb0, gb1)
    sems = (sem0, sem1)
    neg_inf = jnp.float32(-jnp.inf)

    def start(c, slot):
        off = (base + c * _RC) * _B
        pltpu.async_copy(y_hbm.at[pl.ds(off, _CW)], ybufs[slot], sems[slot])
        pltpu.async_copy(m_hbm.at[pl.ds(off, _CW)], mbufs[slot], sems[slot])
        pltpu.async_copy(g_hbm.at[pl.ds(off, _CW)], gbufs[slot], sems[slot])

    def wait(slot):
        pltpu.make_async_copy(y_hbm.at[pl.ds(0, _CW)], ybufs[slot],
                              sems[slot]).wait()
        pltpu.make_async_copy(m_hbm.at[pl.ds(0, _CW)], mbufs[slot],
                              sems[slot]).wait()
        pltpu.make_async_copy(g_hbm.at[pl.ds(0, _CW)], gbufs[slot],
                              sems[slot]).wait()

    carry = ([jnp.full((16,), neg_inf, jnp.float32) for _ in range(8)]
             + [jnp.full((16,), _V, jnp.int32) for _ in range(8)])

    start(0, 0)
    for c in range(_NCH):
        slot = c % 2
        wait(slot)
        if c + 1 < _NCH:
            start(c + 1, slot ^ 1)
        yb, mb, gb = ybufs[slot], mbufs[slot], gbufs[slot]
        vbase = base + c * _RC

        def row_body(r, acc, yb=yb, mb=mb, gb=gb, vbase=vbase):
            acc = list(acc)
            iv = jnp.zeros((16,), jnp.int32) + (vbase + r)
            for gi in range(8):
                w = r * _B + gi * 16
                yv = yb[pl.ds(w, 16)]
                mv = mb[pl.ds(w, 16)]
                gv = gb[pl.ds(w, 16)]
                sc = jnp.where(mv > 0, yv + gv, neg_inf)
                better = sc > acc[gi]
                acc[gi] = jnp.where(better, sc, acc[gi])
                acc[8 + gi] = jnp.where(better, iv, acc[8 + gi])
            return tuple(acc)

        carry = lax.fori_loop(0, _RC, row_body, tuple(carry))

    carry = list(carry)
    for gi in range(8):
        sbuf[pl.ds(gi * 16, 16)] = carry[gi]
        ibuf[pl.ds(gi * 16, 16)] = carry[8 + gi]
    pltpu.sync_copy(sbuf, s_out.at[pl.ds(wid * _B, _B)])
    pltpu.sync_copy(ibuf, i_out.at[pl.ds(wid * _B, _B)])


@functools.partial(
    pl.kernel,
    mesh=plsc.VectorSubcoreMesh(core_axis_name="c", subcore_axis_name="s"),
    out_type=[
        jax.ShapeDtypeStruct((_NW * _B,), jnp.float32),
        jax.ShapeDtypeStruct((_NW * _B,), jnp.int32),
    ],
    scratch_types=[
        pltpu.VMEM((_CW,), jnp.float32),
        pltpu.VMEM((_CW,), jnp.float32),
        pltpu.VMEM((_CW,), jnp.float32),
        pltpu.VMEM((_CW,), jnp.float32),
        pltpu.VMEM((_CW,), jnp.float32),
        pltpu.VMEM((_CW,), jnp.float32),
        pltpu.VMEM((_B,), jnp.float32),
        pltpu.VMEM((_B,), jnp.int32),
        pltpu.SemaphoreType.DMA,
        pltpu.SemaphoreType.DMA,
    ],
)
def _sc_sample(y_hbm, m_hbm, g_hbm, s_out, i_out, *scratch):
    _sc_sample_body(y_hbm, m_hbm, g_hbm, s_out, i_out, *scratch)


def kernel(y_t, ymask):
    y_T = y_t.T
    mask_T = ymask.T

    gp_T = _tc_goldprobs(y_T, mask_T)

    s_flat, i_flat = _sc_sample(y_T.reshape(-1), mask_T.reshape(-1),
                                _GUMBEL_T.reshape(-1))
    scores = s_flat.reshape(_NW, _B)
    idxs = i_flat.reshape(_NW, _B)
    best = jnp.max(scores, axis=0, keepdims=True)
    idx = jnp.min(jnp.where(scores == best, idxs, jnp.int32(_V)), axis=0)
    idx = idx.astype(jnp.int32)
    return (idx, idx, gp_T.T)
